# baseline (device time: 55250 ns/iter reference)
import jax
import jax.numpy as jnp
from jax import lax
from jax.experimental import pallas as pl
from jax.experimental.pallas import tpu as pltpu

N_DEV = 4
SQ = 1024
SKV = 1024
H_LOCAL = 8
DH = 128
D_MODEL = 1024
H_DIM = H_LOCAL * DH
WINDOW = 128
SCALE = 0.08838834764831843
CHUNK = SQ // N_DEV
KV_W = CHUNK + 2 * WINDOW


def kernel(x, Wq, K_ext, V_ext, Wo):
    my_pos = lax.axis_index("i")
    x_bf = x[0].astype(jnp.bfloat16)
    Wq_loc = (lax.dynamic_slice(Wq, (0, my_pos * H_DIM), (D_MODEL, H_DIM))
              * SCALE).astype(jnp.bfloat16)
    Wo_loc = lax.dynamic_slice(
        Wo, (my_pos * H_DIM, 0), (H_DIM, D_MODEL)).astype(jnp.bfloat16)
    K_bf = K_ext[0].astype(jnp.bfloat16)
    V_bf = V_ext[0].astype(jnp.bfloat16)

    def body(x_ref, wq_ref, k_ref, v_ref, wo_ref, out_ref,
             partial_ref, rs_recv_ref, ag_ref,
             rs_send_sems, rs_recv_sems, ag_send_sems, ag_recv_sems):
        me = lax.axis_index("i")

        barrier_sem = pltpu.get_barrier_semaphore()
        for p in range(N_DEV):
            @pl.when(p != me)
            def _():
                pl.semaphore_signal(
                    barrier_sem, inc=1,
                    device_id=(p,), device_id_type=pl.DeviceIdType.MESH,
                )

        for step in range(N_DEV):
            c = (me + 1 + step) % N_DEV
            row0 = c * CHUNK
            lo = jnp.clip(row0 - WINDOW, 0, SKV - KV_W)

            qc = jnp.dot(x_ref[pl.ds(row0, CHUNK), :], wq_ref[:, :],
                         preferred_element_type=jnp.float32)

            qi = lax.broadcasted_iota(jnp.int32, (CHUNK, KV_W), 0) + row0
            ki = lax.broadcasted_iota(jnp.int32, (CHUNK, KV_W), 1) + lo
            mask = jnp.abs(qi - ki) <= WINDOW

            ctxs = []
            for h in range(H_LOCAL):
                q = qc[:, h * DH:(h + 1) * DH].astype(jnp.bfloat16)
                k = k_ref[pl.ds(lo, KV_W), h, :]
                s = lax.dot_general(q, k,
                                    (((1,), (1,)), ((), ())),
                                    preferred_element_type=jnp.float32)
                w = jnp.exp(jnp.where(mask, s, -1e9))
                denom = jnp.sum(w, axis=1, keepdims=True)
                ctx_h = jnp.dot(w.astype(jnp.bfloat16),
                                v_ref[pl.ds(lo, KV_W), h, :],
                                preferred_element_type=jnp.float32)
                ctxs.append(ctx_h * (1.0 / denom))
            ctx = jnp.concatenate(ctxs, axis=1).astype(jnp.bfloat16)
            acc_c = jnp.dot(ctx, wo_ref[:, :], preferred_element_type=jnp.float32)

            if step == 0:
                pl.semaphore_wait(barrier_sem, N_DEV - 1)

            if step < N_DEV - 1:
                partial_ref[step, :, :] = acc_c.astype(jnp.bfloat16)
                pltpu.make_async_remote_copy(
                    src_ref=partial_ref.at[step],
                    dst_ref=rs_recv_ref.at[me],
                    send_sem=rs_send_sems.at[step],
                    recv_sem=rs_recv_sems.at[me],
                    device_id=(c,),
                    device_id_type=pl.DeviceIdType.MESH,
                ).start()
            else:
                for j in range(N_DEV):
                    @pl.when(j == me)
                    def _():
                        rs_recv_ref[j, :, :] = acc_c.astype(jnp.bfloat16)

        for j in range(N_DEV):
            @pl.when(j != me)
            def _():
                pltpu.make_async_remote_copy(
                    src_ref=rs_recv_ref.at[j],
                    dst_ref=rs_recv_ref.at[j],
                    send_sem=rs_send_sems.at[0],
                    recv_sem=rs_recv_sems.at[j],
                    device_id=(j,),
                    device_id_type=pl.DeviceIdType.MESH,
                ).wait_recv()

        red = rs_recv_ref[0, :, :].astype(jnp.float32)
        for j in range(1, N_DEV):
            red = red + rs_recv_ref[j, :, :].astype(jnp.float32)

        for j in range(N_DEV):
            @pl.when(j == me)
            def _():
                ag_ref[j, :, :] = red.astype(jnp.bfloat16)

        for j in range(N_DEV):
            @pl.when(j != me)
            def _():
                pltpu.make_async_remote_copy(
                    src_ref=ag_ref.at[me],
                    dst_ref=ag_ref.at[me],
                    send_sem=ag_send_sems.at[j],
                    recv_sem=ag_recv_sems.at[me],
                    device_id=(j,),
                    device_id_type=pl.DeviceIdType.MESH,
                ).start()

        for j in range(N_DEV):
            @pl.when(j != me)
            def _():
                pltpu.make_async_remote_copy(
                    src_ref=ag_ref.at[j],
                    dst_ref=ag_ref.at[j],
                    send_sem=ag_send_sems.at[j],
                    recv_sem=ag_recv_sems.at[j],
                    device_id=(j,),
                    device_id_type=pl.DeviceIdType.MESH,
                ).wait_recv()

        out_ref[:, :] = ag_ref[:, :, :].astype(jnp.float32).reshape(SQ, D_MODEL)
        for j in range(N_DEV):
            @pl.when(j == me)
            def _():
                out_ref[j * CHUNK:(j + 1) * CHUNK, :] = red

        for s in range(N_DEV - 1):
            pltpu.make_async_remote_copy(
                src_ref=partial_ref.at[s],
                dst_ref=partial_ref.at[s],
                send_sem=rs_send_sems.at[s],
                recv_sem=rs_recv_sems.at[0],
                device_id=(0,),
                device_id_type=pl.DeviceIdType.MESH,
            ).wait_send()
        for j in range(N_DEV):
            @pl.when(j != me)
            def _():
                pltpu.make_async_remote_copy(
                    src_ref=ag_ref.at[0],
                    dst_ref=ag_ref.at[0],
                    send_sem=ag_send_sems.at[j],
                    recv_sem=ag_recv_sems.at[j],
                    device_id=(j,),
                    device_id_type=pl.DeviceIdType.MESH,
                ).wait_send()

    out = pl.pallas_call(
        body,
        out_shape=jax.ShapeDtypeStruct((SQ, D_MODEL), jnp.float32),
        in_specs=[pl.BlockSpec(memory_space=pltpu.VMEM)] * 5,
        out_specs=pl.BlockSpec(memory_space=pltpu.VMEM),
        scratch_shapes=[
            pltpu.VMEM((N_DEV - 1, CHUNK, D_MODEL), jnp.bfloat16),
            pltpu.VMEM((N_DEV, CHUNK, D_MODEL), jnp.bfloat16),
            pltpu.VMEM((N_DEV, CHUNK, D_MODEL), jnp.bfloat16),
            pltpu.SemaphoreType.DMA((N_DEV - 1,)),
            pltpu.SemaphoreType.DMA((N_DEV,)),
            pltpu.SemaphoreType.DMA((N_DEV,)),
            pltpu.SemaphoreType.DMA((N_DEV,)),
        ],
        compiler_params=pltpu.CompilerParams(collective_id=0),
    )(x_bf, Wq_loc, K_bf, V_bf, Wo_loc)
    return out[None]


# device time: 43797 ns/iter; 1.2615x vs baseline; 1.2615x over previous
import jax
import jax.numpy as jnp
from jax import lax
from jax.experimental import pallas as pl
from jax.experimental.pallas import tpu as pltpu

N_DEV = 4
SQ = 1024
SKV = 1024
H_LOCAL = 8
DH = 128
D_MODEL = 1024
H_DIM = H_LOCAL * DH
WINDOW = 128
SCALE = 0.08838834764831843
CHUNK = SQ // N_DEV
KV_W = CHUNK + 2 * WINDOW
HALF = CHUNK // 2


def kernel(x, Wq, K_ext, V_ext, Wo):
    x2 = x[0]
    K2 = jnp.transpose(K_ext[0], (1, 0, 2))
    V2 = jnp.transpose(V_ext[0], (1, 0, 2))

    def body(x_ref, wq_hbm, k_ref, v_ref, wo_hbm, out_ref,
             wq_ref, wo_ref, partial_ref, rs_recv_ref, ag_ref,
             load_sems, rs_send_sems, rs_recv_sems, ag_send_sems, ag_recv_sems):
        me = lax.axis_index("i")

        barrier_sem = pltpu.get_barrier_semaphore()
        for p in range(N_DEV):
            @pl.when(p != me)
            def _():
                pl.semaphore_signal(
                    barrier_sem, inc=1,
                    device_id=(p,), device_id_type=pl.DeviceIdType.MESH,
                )

        wq_cp = pltpu.make_async_copy(
            wq_hbm.at[:, pl.ds(me * H_DIM, H_DIM)], wq_ref, load_sems.at[0])
        wo_cp = pltpu.make_async_copy(
            wo_hbm.at[pl.ds(me * H_DIM, H_DIM), :], wo_ref, load_sems.at[1])
        wq_cp.start()
        wo_cp.start()
        wq_cp.wait()

        for step in range(N_DEV):
            c = (me + 1 + step) % N_DEV
            row0 = c * CHUNK
            lo = jnp.clip(row0 - WINDOW, 0, SKV - KV_W)

            qc = jnp.dot(x_ref[pl.ds(row0, CHUNK), :], wq_ref[:, :],
                         preferred_element_type=jnp.float32)

            qi = lax.broadcasted_iota(jnp.int32, (CHUNK, KV_W), 0) + row0
            ki = lax.broadcasted_iota(jnp.int32, (CHUNK, KV_W), 1) + lo
            mask = jnp.abs(qi - ki) <= WINDOW

            ctxs = []
            for h in range(H_LOCAL):
                q = qc[:, h * DH:(h + 1) * DH]
                k = k_ref[h, pl.ds(lo, KV_W), :]
                s = lax.dot_general(q, k,
                                    (((1,), (1,)), ((), ())),
                                    preferred_element_type=jnp.float32) * SCALE
                w = jnp.exp(jnp.where(mask, s, -1e9))
                denom = jnp.sum(w, axis=1, keepdims=True)
                ctx_h = jnp.dot(w, v_ref[h, pl.ds(lo, KV_W), :],
                                preferred_element_type=jnp.float32)
                ctxs.append(ctx_h * (1.0 / denom))
            ctx = jnp.concatenate(ctxs, axis=1)
            if step == 0:
                wo_cp.wait()
            acc_c = jnp.dot(ctx, wo_ref[:, :], preferred_element_type=jnp.float32)

            if step == 0:
                pl.semaphore_wait(barrier_sem, N_DEV - 1)

            if step < N_DEV - 1:
                partial_ref[step, :, :] = acc_c.astype(jnp.bfloat16)
                pltpu.make_async_remote_copy(
                    src_ref=partial_ref.at[step],
                    dst_ref=rs_recv_ref.at[me],
                    send_sem=rs_send_sems.at[step],
                    recv_sem=rs_recv_sems.at[me],
                    device_id=(c,),
                    device_id_type=pl.DeviceIdType.MESH,
                ).start()
            else:
                for j in range(N_DEV):
                    @pl.when(j == me)
                    def _():
                        rs_recv_ref[j, :, :] = acc_c.astype(jnp.bfloat16)

        for j in range(N_DEV):
            @pl.when(j != me)
            def _():
                pltpu.make_async_remote_copy(
                    src_ref=rs_recv_ref.at[j],
                    dst_ref=rs_recv_ref.at[j],
                    send_sem=rs_send_sems.at[0],
                    recv_sem=rs_recv_sems.at[j],
                    device_id=(j,),
                    device_id_type=pl.DeviceIdType.MESH,
                ).wait_recv()

        reds = []
        for half in range(2):
            r0 = half * HALF
            red = rs_recv_ref[0, pl.ds(r0, HALF), :].astype(jnp.float32)
            for j in range(1, N_DEV):
                red = red + rs_recv_ref[j, pl.ds(r0, HALF), :].astype(jnp.float32)
            reds.append(red)
            for j in range(N_DEV):
                @pl.when(j == me)
                def _():
                    ag_ref[j, half, :, :] = red.astype(jnp.bfloat16)
            for j in range(N_DEV):
                @pl.when(j != me)
                def _():
                    pltpu.make_async_remote_copy(
                        src_ref=ag_ref.at[me, half],
                        dst_ref=ag_ref.at[me, half],
                        send_sem=ag_send_sems.at[j, half],
                        recv_sem=ag_recv_sems.at[me, half],
                        device_id=(j,),
                        device_id_type=pl.DeviceIdType.MESH,
                    ).start()

        for half in range(2):
            for j in range(N_DEV):
                @pl.when(j != me)
                def _():
                    pltpu.make_async_remote_copy(
                        src_ref=ag_ref.at[j, half],
                        dst_ref=ag_ref.at[j, half],
                        send_sem=ag_send_sems.at[j, half],
                        recv_sem=ag_recv_sems.at[j, half],
                        device_id=(j,),
                        device_id_type=pl.DeviceIdType.MESH,
                    ).wait_recv()
                    out_ref[j * CHUNK + half * HALF:
                            j * CHUNK + (half + 1) * HALF, :] = (
                        ag_ref[j, half, :, :].astype(jnp.float32))
        for j in range(N_DEV):
            @pl.when(j == me)
            def _():
                out_ref[j * CHUNK:j * CHUNK + HALF, :] = reds[0]
                out_ref[j * CHUNK + HALF:(j + 1) * CHUNK, :] = reds[1]

        for s in range(N_DEV - 1):
            pltpu.make_async_remote_copy(
                src_ref=partial_ref.at[s],
                dst_ref=partial_ref.at[s],
                send_sem=rs_send_sems.at[s],
                recv_sem=rs_recv_sems.at[0],
                device_id=(0,),
                device_id_type=pl.DeviceIdType.MESH,
            ).wait_send()
        for half in range(2):
            for j in range(N_DEV):
                @pl.when(j != me)
                def _():
                    pltpu.make_async_remote_copy(
                        src_ref=ag_ref.at[0, half],
                        dst_ref=ag_ref.at[0, half],
                        send_sem=ag_send_sems.at[j, half],
                        recv_sem=ag_recv_sems.at[j, half],
                        device_id=(j,),
                        device_id_type=pl.DeviceIdType.MESH,
                    ).wait_send()

    out = pl.pallas_call(
        body,
        out_shape=jax.ShapeDtypeStruct((SQ, D_MODEL), jnp.float32),
        in_specs=[
            pl.BlockSpec(memory_space=pltpu.VMEM),
            pl.BlockSpec(memory_space=pltpu.MemorySpace.HBM),
            pl.BlockSpec(memory_space=pltpu.VMEM),
            pl.BlockSpec(memory_space=pltpu.VMEM),
            pl.BlockSpec(memory_space=pltpu.MemorySpace.HBM),
        ],
        out_specs=pl.BlockSpec(memory_space=pltpu.VMEM),
        scratch_shapes=[
            pltpu.VMEM((D_MODEL, H_DIM), jnp.float32),
            pltpu.VMEM((H_DIM, D_MODEL), jnp.float32),
            pltpu.VMEM((N_DEV - 1, CHUNK, D_MODEL), jnp.bfloat16),
            pltpu.VMEM((N_DEV, CHUNK, D_MODEL), jnp.bfloat16),
            pltpu.VMEM((N_DEV, 2, HALF, D_MODEL), jnp.bfloat16),
            pltpu.SemaphoreType.DMA((2,)),
            pltpu.SemaphoreType.DMA((N_DEV - 1,)),
            pltpu.SemaphoreType.DMA((N_DEV,)),
            pltpu.SemaphoreType.DMA((N_DEV, 2)),
            pltpu.SemaphoreType.DMA((N_DEV, 2)),
        ],
        compiler_params=pltpu.CompilerParams(collective_id=0),
    )(x2, Wq, K2, V2, Wo)
    return out[None]


# device time: 43772 ns/iter; 1.2622x vs baseline; 1.0006x over previous
import jax
import jax.numpy as jnp
from jax import lax
from jax.experimental import pallas as pl
from jax.experimental.pallas import tpu as pltpu

N_DEV = 4
SQ = 1024
SKV = 1024
H_LOCAL = 8
DH = 128
D_MODEL = 1024
H_DIM = H_LOCAL * DH
WINDOW = 128
SCALE = 0.08838834764831843
CHUNK = SQ // N_DEV
KV_W = CHUNK + 2 * WINDOW
NQ = 4
QR = CHUNK // NQ


def kernel(x, Wq, K_ext, V_ext, Wo):
    x2 = x[0]
    K2 = jnp.transpose(K_ext[0], (1, 0, 2))
    V2 = jnp.transpose(V_ext[0], (1, 0, 2))

    def body(x_ref, wq_hbm, k_ref, v_ref, wo_hbm, out_ref,
             wq_ref, wo_ref, partial_ref, rs_recv_ref, ag_ref,
             load_sems, rs_send_sems, rs_recv_sems, ag_send_sems, ag_recv_sems):
        me = lax.axis_index("i")

        barrier_sem = pltpu.get_barrier_semaphore()
        for p in range(N_DEV):
            @pl.when(p != me)
            def _():
                pl.semaphore_signal(
                    barrier_sem, inc=1,
                    device_id=(p,), device_id_type=pl.DeviceIdType.MESH,
                )

        wq_cp = pltpu.make_async_copy(
            wq_hbm.at[:, pl.ds(me * H_DIM, H_DIM)], wq_ref, load_sems.at[0])
        wo_cp = pltpu.make_async_copy(
            wo_hbm.at[pl.ds(me * H_DIM, H_DIM), :], wo_ref, load_sems.at[1])
        wq_cp.start()
        wo_cp.start()
        wq_cp.wait()

        for step in range(N_DEV):
            c = (me + 1 + step) % N_DEV
            row0 = c * CHUNK
            lo = jnp.clip(row0 - WINDOW, 0, SKV - KV_W)

            qc = jnp.dot(x_ref[pl.ds(row0, CHUNK), :], wq_ref[:, :],
                         preferred_element_type=jnp.float32)

            qi = lax.broadcasted_iota(jnp.int32, (CHUNK, KV_W), 0) + row0
            ki = lax.broadcasted_iota(jnp.int32, (CHUNK, KV_W), 1) + lo
            mask = jnp.abs(qi - ki) <= WINDOW

            ctxs = []
            for h in range(H_LOCAL):
                q = qc[:, h * DH:(h + 1) * DH]
                k = k_ref[h, pl.ds(lo, KV_W), :]
                s = lax.dot_general(q, k,
                                    (((1,), (1,)), ((), ())),
                                    preferred_element_type=jnp.float32) * SCALE
                w = jnp.exp(jnp.where(mask, s, -1e9))
                denom = jnp.sum(w, axis=1, keepdims=True)
                ctx_h = jnp.dot(w, v_ref[h, pl.ds(lo, KV_W), :],
                                preferred_element_type=jnp.float32)
                ctxs.append(ctx_h * (1.0 / denom))
            ctx = jnp.concatenate(ctxs, axis=1)
            if step == 0:
                wo_cp.wait()
            acc_c = jnp.dot(ctx, wo_ref[:, :], preferred_element_type=jnp.float32)

            if step == 0:
                pl.semaphore_wait(barrier_sem, N_DEV - 1)

            if step < N_DEV - 1:
                partial_ref[step, :, :] = acc_c.astype(jnp.bfloat16)
                pltpu.make_async_remote_copy(
                    src_ref=partial_ref.at[step],
                    dst_ref=rs_recv_ref.at[me],
                    send_sem=rs_send_sems.at[step],
                    recv_sem=rs_recv_sems.at[me],
                    device_id=(c,),
                    device_id_type=pl.DeviceIdType.MESH,
                ).start()
            else:
                for j in range(N_DEV):
                    @pl.when(j == me)
                    def _():
                        rs_recv_ref[j, :, :] = acc_c.astype(jnp.bfloat16)

        for j in range(N_DEV):
            @pl.when(j != me)
            def _():
                pltpu.make_async_remote_copy(
                    src_ref=rs_recv_ref.at[j],
                    dst_ref=rs_recv_ref.at[j],
                    send_sem=rs_send_sems.at[0],
                    recv_sem=rs_recv_sems.at[j],
                    device_id=(j,),
                    device_id_type=pl.DeviceIdType.MESH,
                ).wait_recv()

        reds = []
        for qt in range(NQ):
            r0 = qt * QR
            red = rs_recv_ref[0, pl.ds(r0, QR), :].astype(jnp.float32)
            for j in range(1, N_DEV):
                red = red + rs_recv_ref[j, pl.ds(r0, QR), :].astype(jnp.float32)
            reds.append(red)
            for j in range(N_DEV):
                @pl.when(j == me)
                def _():
                    ag_ref[j, qt, :, :] = red.astype(jnp.bfloat16)
            for j in range(N_DEV):
                @pl.when(j != me)
                def _():
                    pltpu.make_async_remote_copy(
                        src_ref=ag_ref.at[me, qt],
                        dst_ref=ag_ref.at[me, qt],
                        send_sem=ag_send_sems.at[j, qt],
                        recv_sem=ag_recv_sems.at[me, qt],
                        device_id=(j,),
                        device_id_type=pl.DeviceIdType.MESH,
                    ).start()

        for qt in range(NQ):
            for j in range(N_DEV):
                @pl.when(j != me)
                def _():
                    pltpu.make_async_remote_copy(
                        src_ref=ag_ref.at[j, qt],
                        dst_ref=ag_ref.at[j, qt],
                        send_sem=ag_send_sems.at[j, qt],
                        recv_sem=ag_recv_sems.at[j, qt],
                        device_id=(j,),
                        device_id_type=pl.DeviceIdType.MESH,
                    ).wait_recv()
                    out_ref[j * CHUNK + qt * QR:
                            j * CHUNK + (qt + 1) * QR, :] = (
                        ag_ref[j, qt, :, :].astype(jnp.float32))
        for j in range(N_DEV):
            @pl.when(j == me)
            def _():
                for qt in range(NQ):
                    out_ref[j * CHUNK + qt * QR:
                            j * CHUNK + (qt + 1) * QR, :] = reds[qt]

        for s in range(N_DEV - 1):
            pltpu.make_async_remote_copy(
                src_ref=partial_ref.at[s],
                dst_ref=partial_ref.at[s],
                send_sem=rs_send_sems.at[s],
                recv_sem=rs_recv_sems.at[0],
                device_id=(0,),
                device_id_type=pl.DeviceIdType.MESH,
            ).wait_send()
        for qt in range(NQ):
            for j in range(N_DEV):
                @pl.when(j != me)
                def _():
                    pltpu.make_async_remote_copy(
                        src_ref=ag_ref.at[0, qt],
                        dst_ref=ag_ref.at[0, qt],
                        send_sem=ag_send_sems.at[j, qt],
                        recv_sem=ag_recv_sems.at[j, qt],
                        device_id=(j,),
                        device_id_type=pl.DeviceIdType.MESH,
                    ).wait_send()

    out = pl.pallas_call(
        body,
        out_shape=jax.ShapeDtypeStruct((SQ, D_MODEL), jnp.float32),
        in_specs=[
            pl.BlockSpec(memory_space=pltpu.VMEM),
            pl.BlockSpec(memory_space=pltpu.MemorySpace.HBM),
            pl.BlockSpec(memory_space=pltpu.VMEM),
            pl.BlockSpec(memory_space=pltpu.VMEM),
            pl.BlockSpec(memory_space=pltpu.MemorySpace.HBM),
        ],
        out_specs=pl.BlockSpec(memory_space=pltpu.VMEM),
        scratch_shapes=[
            pltpu.VMEM((D_MODEL, H_DIM), jnp.float32),
            pltpu.VMEM((H_DIM, D_MODEL), jnp.float32),
            pltpu.VMEM((N_DEV - 1, CHUNK, D_MODEL), jnp.bfloat16),
            pltpu.VMEM((N_DEV, CHUNK, D_MODEL), jnp.bfloat16),
            pltpu.VMEM((N_DEV, NQ, QR, D_MODEL), jnp.bfloat16),
            pltpu.SemaphoreType.DMA((2,)),
            pltpu.SemaphoreType.DMA((N_DEV - 1,)),
            pltpu.SemaphoreType.DMA((N_DEV,)),
            pltpu.SemaphoreType.DMA((N_DEV, NQ)),
            pltpu.SemaphoreType.DMA((N_DEV, NQ)),
        ],
        compiler_params=pltpu.CompilerParams(collective_id=0),
    )(x2, Wq, K2, V2, Wo)
    return out[None]
